# Initial kernel scaffold; baseline (speedup 1.0000x reference)
import jax
import jax.numpy as jnp
from jax.experimental import pallas as pl


def kernel(similarities_matrix, signs_matrix):
    raise NotImplementedError("write your pallas kernel here")


# speedup vs baseline: 4.8679x; 4.8679x over previous
import functools

import numpy as np
import jax
import jax.numpy as jnp
from jax import lax
from jax.experimental import pallas as pl
from jax.experimental.pallas import tpu as pltpu
from jax.experimental.pallas import tpu_sc as plsc

_R = 151
_DELTA = 2.0 / (float(_R) - 1.0)
_D32 = np.float32(_DELTA)
_N = 512
_NW = 32
_ROWS_PER_W = _N // _NW
_ELEMS_PER_W = _ROWS_PER_W * _N
_NBIN = 256
_BSTRIDE = 160
_CSTRIDE = 2 * _BSTRIDE
_ACC = 16 * _CSTRIDE


def _build_bilinear() -> np.ndarray:
    size = max(0, int(np.ceil((1.0 - (-1.0)) / _DELTA)))
    t = (np.float32(-1.0)
         + _D32 * np.arange(size, dtype=np.float32)).astype(np.float32)
    tsize = t.shape[0]
    tm = (t - _D32).astype(np.float32)
    g = np.zeros((tsize, 2 * _NBIN), np.float64)
    for m in range(_NBIN):
        d = np.float32(np.float32(np.float32(m) * _D32) - np.float32(1.0))
        hb = np.nonzero(t == d)[0]
        ha = np.nonzero(tm == d)[0]
        if len(hb):
            kb = hb[0]
            tv = np.float64(t[kb])
            g[kb, m] += (tv + _DELTA) / _DELTA
            g[kb, _NBIN + m] += -1.0 / _DELTA
        if len(ha):
            ka = ha[0]
            tv = np.float64(t[ka])
            g[ka, m] += (_DELTA - tv) / _DELTA
            g[ka, _NBIN + m] += 1.0 / _DELTA
    lower = np.tril(np.ones((tsize, tsize)))
    return (g.T @ lower @ g).astype(np.float32)


_M_CONST = _build_bilinear()

_mesh = plsc.VectorSubcoreMesh(core_axis_name="c", subcore_axis_name="s")


@functools.partial(
    pl.kernel,
    out_type=jax.ShapeDtypeStruct((_NW * 1024,), jnp.float32),
    mesh=_mesh,
    scratch_types=[
        pltpu.VMEM((_ELEMS_PER_W,), jnp.float32),
        pltpu.VMEM((_ELEMS_PER_W,), jnp.int32),
        pltpu.VMEM((_ACC,), jnp.float32),
        pltpu.VMEM((_ACC,), jnp.float32),
        pltpu.VMEM((1024,), jnp.float32),
    ],
    compiler_params=pltpu.CompilerParams(needs_layout_passes=False),
)
def _sc_hist(sims_hbm, signs_hbm, out_hbm, srows, grows, acc_c, acc_s, stage):
    wid = lax.axis_index("s") * 2 + lax.axis_index("c")
    base = wid * _ELEMS_PER_W
    pltpu.sync_copy(sims_hbm.at[pl.ds(base, _ELEMS_PER_W)], srows)
    pltpu.sync_copy(signs_hbm.at[pl.ds(base, _ELEMS_PER_W)], grows)

    zero = jnp.zeros((16,), jnp.float32)
    ones = jnp.full((16,), 1.0, jnp.float32)
    lane = lax.iota(jnp.int32, 16)

    def zbody(k, carry):
        acc_c[pl.ds(k * 16, 16)] = zero
        acc_s[pl.ds(k * 16, 16)] = zero
        return carry

    lax.fori_loop(0, _ACC // 16, zbody, 0)

    row0 = wid * _ROWS_PER_W
    nchunk = _N // 16

    def body(i, carry):
        s = srows[pl.ds(i * 16, 16)]
        g = grows[pl.ds(i * 16, 16)]
        r = row0 + i // nchunk
        col = lane + (i % nchunk) * 16
        mask = col > r
        q = (s + 1.0) / _D32
        mi = jnp.clip(q.astype(jnp.int32), 0, _BSTRIDE - 1)
        idx = lane * _CSTRIDE + g * _BSTRIDE + mi
        plsc.addupdate_scatter(acc_c, [idx], ones, mask=mask)
        plsc.addupdate_scatter(acc_s, [idx], s, mask=mask)
        return carry

    lax.fori_loop(0, _ROWS_PER_W * nchunk, body, 0)

    for k in range(1024 // 16):
        stage[pl.ds(k * 16, 16)] = zero
    for accr, obase in ((acc_c, 0), (acc_s, 512)):
        for chan in range(2):
            for chunk in range(_BSTRIDE // 16):
                v = zero
                for ln in range(16):
                    v = v + accr[pl.ds(ln * _CSTRIDE + chan * _BSTRIDE + chunk * 16, 16)]
                stage[pl.ds(obase + chan * _NBIN + chunk * 16, 16)] = v
    pltpu.sync_copy(stage, out_hbm.at[pl.ds(wid * 1024, 1024)])


def _tc_body(p_ref, m_ref, o_ref):
    x = jnp.sum(p_ref[...], axis=0, keepdims=True)
    xn = jnp.concatenate([x[:, 0:256], x[:, 512:768]], axis=1)
    xp = jnp.concatenate([x[:, 256:512], x[:, 768:1024]], axis=1)
    ns = jnp.sum(x[:, 0:256])
    ps = jnp.sum(x[:, 256:512])
    xn8 = jnp.broadcast_to(xn, (8, 512))
    t8 = jnp.dot(xn8, m_ref[...], preferred_element_type=jnp.float32,
                 precision=jax.lax.Precision.HIGHEST)
    lu = jnp.sum(t8 * jnp.broadcast_to(xp, (8, 512))) * 0.125
    o_ref[...] = (lu / (ps * ns)).reshape(1, 1)


_finalize = pl.pallas_call(
    _tc_body,
    out_shape=jax.ShapeDtypeStruct((1, 1), jnp.float32),
)


def kernel(similarities_matrix, signs_matrix):
    s = similarities_matrix.reshape(-1)
    g = signs_matrix.astype(jnp.int32).reshape(-1)
    partials = _sc_hist(s, g)
    out = _finalize(partials.reshape(_NW, 1024), _M_CONST)
    return out.reshape(())


# speedup vs baseline: 5.2152x; 1.0714x over previous
import functools

import numpy as np
import jax
import jax.numpy as jnp
from jax import lax
from jax.experimental import pallas as pl
from jax.experimental.pallas import tpu as pltpu
from jax.experimental.pallas import tpu_sc as plsc

_R = 151
_DELTA = 2.0 / (float(_R) - 1.0)
_D32 = np.float32(_DELTA)
_N = 512
_NW = 32
_BLK = _N // (2 * _NW)
_BLKE = _BLK * _N
_NBIN = 256
_BSTRIDE = 160
_CSTRIDE = 2 * _BSTRIDE
_ACC = 16 * _CSTRIDE


def _build_bilinear() -> np.ndarray:
    size = max(0, int(np.ceil((1.0 - (-1.0)) / _DELTA)))
    t = (np.float32(-1.0)
         + _D32 * np.arange(size, dtype=np.float32)).astype(np.float32)
    tsize = t.shape[0]
    tm = (t - _D32).astype(np.float32)
    g = np.zeros((tsize, 2 * _NBIN), np.float64)
    for m in range(_NBIN):
        d = np.float32(np.float32(np.float32(m) * _D32) - np.float32(1.0))
        hb = np.nonzero(t == d)[0]
        ha = np.nonzero(tm == d)[0]
        if len(hb):
            kb = hb[0]
            tv = np.float64(t[kb])
            g[kb, m] += (tv + _DELTA) / _DELTA
            g[kb, _NBIN + m] += -1.0 / _DELTA
        if len(ha):
            ka = ha[0]
            tv = np.float64(t[ka])
            g[ka, m] += (_DELTA - tv) / _DELTA
            g[ka, _NBIN + m] += 1.0 / _DELTA
    lower = np.tril(np.ones((tsize, tsize)))
    return (g.T @ lower @ g).astype(np.float32)


_M_CONST = _build_bilinear()
_ZEROS = np.zeros((_ACC,), np.float32)

_mesh = plsc.VectorSubcoreMesh(core_axis_name="c", subcore_axis_name="s")


@functools.partial(
    pl.kernel,
    out_type=jax.ShapeDtypeStruct((_NW * 1024,), jnp.float32),
    mesh=_mesh,
    scratch_types=[
        pltpu.VMEM((2 * _BLKE,), jnp.float32),
        pltpu.VMEM((2 * _BLKE,), jnp.int32),
        pltpu.VMEM((_ACC,), jnp.float32),
        pltpu.VMEM((_ACC,), jnp.float32),
        pltpu.VMEM((1024,), jnp.float32),
        pltpu.SemaphoreType.DMA,
    ],
    compiler_params=pltpu.CompilerParams(needs_layout_passes=False),
)
def _sc_hist(sims_hbm, signs_hbm, zeros_hbm, out_hbm,
             srows, grows, acc_c, acc_s, stage, sem):
    wid = lax.axis_index("s") * 2 + lax.axis_index("c")
    r0a = _BLK * wid
    r0b = _N - _BLK * (wid + 1)
    basea = r0a * _N
    baseb = r0b * _N
    copies = [
        pltpu.async_copy(sims_hbm.at[pl.ds(basea, _BLKE)],
                         srows.at[pl.ds(0, _BLKE)], sem),
        pltpu.async_copy(sims_hbm.at[pl.ds(baseb, _BLKE)],
                         srows.at[pl.ds(_BLKE, _BLKE)], sem),
        pltpu.async_copy(signs_hbm.at[pl.ds(basea, _BLKE)],
                         grows.at[pl.ds(0, _BLKE)], sem),
        pltpu.async_copy(signs_hbm.at[pl.ds(baseb, _BLKE)],
                         grows.at[pl.ds(_BLKE, _BLKE)], sem),
        pltpu.async_copy(zeros_hbm, acc_c, sem),
        pltpu.async_copy(zeros_hbm, acc_s, sem),
    ]
    for c in copies:
        c.wait()

    zero = jnp.zeros((16,), jnp.float32)
    ones = jnp.full((16,), 1.0, jnp.float32)
    lane = lax.iota(jnp.int32, 16)
    lane_off = lane * _CSTRIDE

    def make_body(buf_base, r0):
        def body(j, carry):
            col = lane + j * 16
            for k in range(_BLK):
                off = buf_base + k * _N + j * 16
                s = srows[pl.ds(off, 16)]
                g = grows[pl.ds(off, 16)]
                mask = col > (r0 + k)
                q = (s + 1.0) / _D32
                mi = jnp.clip(q.astype(jnp.int32), 0, _BSTRIDE - 1)
                idx = lane_off + g * _BSTRIDE + mi
                plsc.addupdate_scatter(acc_c, [idx], ones, mask=mask)
                plsc.addupdate_scatter(acc_s, [idx], s, mask=mask)
            return carry
        return body

    lax.fori_loop(r0a // 16, _N // 16, make_body(0, r0a), 0)
    lax.fori_loop(r0b // 16, _N // 16, make_body(_BLKE, r0b), 0)

    for k in range(1024 // 16):
        stage[pl.ds(k * 16, 16)] = zero

    def make_red(accr, sum_sel):
        def red(c, carry):
            chan = c // 10
            chunk = c - chan * 10
            v = zero
            for ln in range(16):
                v = v + accr[pl.ds(ln * _CSTRIDE + chan * _BSTRIDE
                                   + chunk * 16, 16)]
            stage[pl.ds(chan * 512 + sum_sel * 256 + chunk * 16, 16)] = v
            return carry
        return red

    lax.fori_loop(0, 20, make_red(acc_c, 0), 0)
    lax.fori_loop(0, 20, make_red(acc_s, 1), 0)
    pltpu.sync_copy(stage, out_hbm.at[pl.ds(wid * 1024, 1024)])


def _tc_body(p_ref, m_ref, o_ref):
    x = jnp.sum(p_ref[...].reshape(_NW, 1024), axis=0)
    xn = x[0:512].reshape(1, 512)
    xp = x[512:1024].reshape(1, 512)
    ns = jnp.sum(x[0:256])
    ps = jnp.sum(x[512:768])
    xn8 = jnp.broadcast_to(xn, (8, 512))
    t8 = jnp.dot(xn8, m_ref[...], preferred_element_type=jnp.float32,
                 precision=jax.lax.Precision.HIGHEST)
    lu = jnp.sum(t8 * jnp.broadcast_to(xp, (8, 512))) * 0.125
    o_ref[...] = (lu / (ps * ns)).reshape(1, 1)


_finalize = pl.pallas_call(
    _tc_body,
    out_shape=jax.ShapeDtypeStruct((1, 1), jnp.float32),
)


def kernel(similarities_matrix, signs_matrix):
    s = similarities_matrix.reshape(-1)
    g = signs_matrix.astype(jnp.int32).reshape(-1)
    partials = _sc_hist(s, g, _ZEROS)
    out = _finalize(partials, _M_CONST)
    return out.reshape(())


# speedup vs baseline: 5.2293x; 1.0027x over previous
import functools

import numpy as np
import jax
import jax.numpy as jnp
from jax import lax
from jax.experimental import pallas as pl
from jax.experimental.pallas import tpu as pltpu
from jax.experimental.pallas import tpu_sc as plsc

_R = 151
_DELTA = 2.0 / (float(_R) - 1.0)
_D32 = np.float32(_DELTA)
_N = 512
_NW = 32
_BLK = _N // (2 * _NW)
_BLKE = _BLK * _N
_NBIN = 256
_BSTRIDE = 160
_CSTRIDE = 2 * _BSTRIDE
_ACC = 16 * _CSTRIDE


def _build_bilinear() -> np.ndarray:
    size = max(0, int(np.ceil((1.0 - (-1.0)) / _DELTA)))
    t = (np.float32(-1.0)
         + _D32 * np.arange(size, dtype=np.float32)).astype(np.float32)
    tsize = t.shape[0]
    tm = (t - _D32).astype(np.float32)
    g = np.zeros((tsize, 2 * _NBIN), np.float64)
    for m in range(_NBIN):
        d = np.float32(np.float32(np.float32(m) * _D32) - np.float32(1.0))
        hb = np.nonzero(t == d)[0]
        ha = np.nonzero(tm == d)[0]
        if len(hb):
            kb = hb[0]
            tv = np.float64(t[kb])
            g[kb, m] += (tv + _DELTA) / _DELTA
            g[kb, _NBIN + m] += -1.0 / _DELTA
        if len(ha):
            ka = ha[0]
            tv = np.float64(t[ka])
            g[ka, m] += (_DELTA - tv) / _DELTA
            g[ka, _NBIN + m] += 1.0 / _DELTA
    lower = np.tril(np.ones((tsize, tsize)))
    return (g.T @ lower @ g).astype(np.float32)


_M_CONST = _build_bilinear()
_ZEROS = np.zeros((_ACC,), np.float32)

_mesh = plsc.VectorSubcoreMesh(core_axis_name="c", subcore_axis_name="s")


@functools.partial(
    pl.kernel,
    out_type=jax.ShapeDtypeStruct((_NW * 1024,), jnp.float32),
    mesh=_mesh,
    scratch_types=[
        pltpu.VMEM((2 * _BLKE,), jnp.float32),
        pltpu.VMEM((2 * _BLKE,), jnp.int32),
        pltpu.VMEM((_ACC,), jnp.float32),
        pltpu.VMEM((_ACC,), jnp.float32),
        pltpu.VMEM((1024,), jnp.float32),
        pltpu.SemaphoreType.DMA,
    ],
    compiler_params=pltpu.CompilerParams(needs_layout_passes=False),
)
def _sc_hist(sims_hbm, signs_hbm, zeros_hbm, out_hbm,
             srows, grows, acc_c, acc_s, stage, sem):
    wid = lax.axis_index("s") * 2 + lax.axis_index("c")
    r0a = _BLK * wid
    r0b = _N - _BLK * (wid + 1)
    basea = r0a * _N
    baseb = r0b * _N
    copies = [
        pltpu.async_copy(sims_hbm.at[pl.ds(basea, _BLKE)],
                         srows.at[pl.ds(0, _BLKE)], sem),
        pltpu.async_copy(sims_hbm.at[pl.ds(baseb, _BLKE)],
                         srows.at[pl.ds(_BLKE, _BLKE)], sem),
        pltpu.async_copy(signs_hbm.at[pl.ds(basea, _BLKE)],
                         grows.at[pl.ds(0, _BLKE)], sem),
        pltpu.async_copy(signs_hbm.at[pl.ds(baseb, _BLKE)],
                         grows.at[pl.ds(_BLKE, _BLKE)], sem),
        pltpu.async_copy(zeros_hbm, acc_c, sem),
        pltpu.async_copy(zeros_hbm, acc_s, sem),
    ]
    for c in copies:
        c.wait()

    zero = jnp.zeros((16,), jnp.float32)
    ones = jnp.full((16,), 1.0, jnp.float32)
    lane = lax.iota(jnp.int32, 16)
    lane_off = lane * _CSTRIDE

    def make_body(buf_base, r0):
        def body(j, carry):
            col = lane + j * 16
            for k in range(_BLK):
                off = buf_base + k * _N + j * 16
                s = srows[pl.ds(off, 16)]
                g = grows[pl.ds(off, 16)]
                mask = col > (r0 + k)
                q = s * np.float32(75.0) + np.float32(75.0)
                mi = jnp.clip(q.astype(jnp.int32), 0, _BSTRIDE - 1)
                idx = lane_off + g * _BSTRIDE + mi
                plsc.addupdate_scatter(acc_c, [idx], ones, mask=mask)
                plsc.addupdate_scatter(acc_s, [idx], s, mask=mask)
            return carry
        return body

    lax.fori_loop(r0a // 16, _N // 16, make_body(0, r0a), 0)
    lax.fori_loop(r0b // 16, _N // 16, make_body(_BLKE, r0b), 0)

    for k in range(1024 // 16):
        stage[pl.ds(k * 16, 16)] = zero

    def make_red(accr, sum_sel):
        def red(c, carry):
            chan = c // 10
            chunk = c - chan * 10
            v = zero
            for ln in range(16):
                v = v + accr[pl.ds(ln * _CSTRIDE + chan * _BSTRIDE
                                   + chunk * 16, 16)]
            stage[pl.ds(chan * 512 + sum_sel * 256 + chunk * 16, 16)] = v
            return carry
        return red

    lax.fori_loop(0, 20, make_red(acc_c, 0), 0)
    lax.fori_loop(0, 20, make_red(acc_s, 1), 0)
    pltpu.sync_copy(stage, out_hbm.at[pl.ds(wid * 1024, 1024)])


def _tc_body(p_ref, m_ref, o_ref):
    x = jnp.sum(p_ref[...].reshape(_NW, 1024), axis=0)
    xn = x[0:512].reshape(1, 512)
    xp = x[512:1024].reshape(1, 512)
    ns = jnp.sum(x[0:256])
    ps = jnp.sum(x[512:768])
    xn8 = jnp.broadcast_to(xn, (8, 512))
    t8 = jnp.dot(xn8, m_ref[...], preferred_element_type=jnp.float32,
                 precision=jax.lax.Precision.HIGHEST)
    lu = jnp.sum(t8 * jnp.broadcast_to(xp, (8, 512))) * 0.125
    o_ref[...] = (lu / (ps * ns)).reshape(1, 1)


_finalize = pl.pallas_call(
    _tc_body,
    out_shape=jax.ShapeDtypeStruct((1, 1), jnp.float32),
)


def kernel(similarities_matrix, signs_matrix):
    s = similarities_matrix.reshape(-1)
    g = signs_matrix.astype(jnp.int32).reshape(-1)
    partials = _sc_hist(s, g, _ZEROS)
    out = _finalize(partials, _M_CONST)
    return out.reshape(())


# speedup vs baseline: 6.1135x; 1.1691x over previous
import functools

import numpy as np
import jax
import jax.numpy as jnp
from jax import lax
from jax.experimental import pallas as pl
from jax.experimental.pallas import tpu as pltpu
from jax.experimental.pallas import tpu_sc as plsc

_R = 151
_DELTA = 2.0 / (float(_R) - 1.0)
_D32 = np.float32(_DELTA)
_N = 512
_NW = 32
_BLK = _N // (2 * _NW)
_BLKE = _BLK * _N
_NBIN = 256
_BSTRIDE = 160
_CSTRIDE = 2 * _BSTRIDE
_ACC = 16 * _CSTRIDE
_QBITS = 12
_MARK = 1 << 22


def _build_bilinear() -> np.ndarray:
    size = max(0, int(np.ceil((1.0 - (-1.0)) / _DELTA)))
    t = (np.float32(-1.0)
         + _D32 * np.arange(size, dtype=np.float32)).astype(np.float32)
    tsize = t.shape[0]
    tm = (t - _D32).astype(np.float32)
    g = np.zeros((tsize, 2 * _NBIN), np.float64)
    for m in range(_NBIN):
        d = np.float32(np.float32(np.float32(m) * _D32) - np.float32(1.0))
        hb = np.nonzero(t == d)[0]
        ha = np.nonzero(tm == d)[0]
        if len(hb):
            kb = hb[0]
            tv = np.float64(t[kb])
            g[kb, m] += (tv + _DELTA) / _DELTA
            g[kb, _NBIN + m] += -1.0 / _DELTA
        if len(ha):
            ka = ha[0]
            tv = np.float64(t[ka])
            g[ka, m] += (_DELTA - tv) / _DELTA
            g[ka, _NBIN + m] += 1.0 / _DELTA
    lower = np.tril(np.ones((tsize, tsize)))
    return (g.T @ lower @ g).astype(np.float32)


_M_CONST = _build_bilinear()
_ZEROS = np.zeros((_ACC,), np.int32)

_mesh = plsc.VectorSubcoreMesh(core_axis_name="c", subcore_axis_name="s")


@functools.partial(
    pl.kernel,
    out_type=jax.ShapeDtypeStruct((_NW * 1024,), jnp.float32),
    mesh=_mesh,
    scratch_types=[
        pltpu.VMEM((2 * _BLKE,), jnp.float32),
        pltpu.VMEM((_ACC,), jnp.int32),
        pltpu.VMEM((1024,), jnp.float32),
        pltpu.SemaphoreType.DMA,
        pltpu.SemaphoreType.DMA,
    ],
    compiler_params=pltpu.CompilerParams(needs_layout_passes=False),
)
def _sc_hist(vals_hbm, zeros_hbm, out_hbm, srows, acc, stage, sema, semb):
    wid = lax.axis_index("s") * 2 + lax.axis_index("c")
    r0a = _BLK * wid
    r0b = _N - _BLK * (wid + 1)
    ca = pltpu.async_copy(vals_hbm.at[pl.ds(r0a * _N, _BLKE)],
                          srows.at[pl.ds(0, _BLKE)], sema)
    cz = pltpu.async_copy(zeros_hbm, acc, sema)
    cb = pltpu.async_copy(vals_hbm.at[pl.ds(r0b * _N, _BLKE)],
                          srows.at[pl.ds(_BLKE, _BLKE)], semb)
    ca.wait()
    cz.wait()

    zero = jnp.zeros((16,), jnp.float32)
    lane = lax.iota(jnp.int32, 16)
    lane_off = lane * _CSTRIDE
    fma_c = np.float32(1 << _QBITS)
    fma_b = np.float32((1 << 22) + 0.5)

    def make_body(buf_base, r0):
        def body(j, carry):
            col = lane + j * 16
            for k in range(_BLK):
                off = buf_base + k * _N + j * 16
                v = srows[pl.ds(off, 16)]
                mask = col > (r0 + k)
                gi = lax.shift_right_logical(
                    plsc.bitcast(v, jnp.int32), 31)
                s = jnp.abs(v) - 3.0
                q = s * np.float32(75.0) + np.float32(75.0)
                mi = jnp.clip(q.astype(jnp.int32), 0, _BSTRIDE - 1)
                idx = lane_off + gi * _BSTRIDE + mi
                packed = (s * fma_c + fma_b).astype(jnp.int32)
                plsc.addupdate_scatter(acc, [idx], packed, mask=mask)
            return carry
        return body

    lax.fori_loop(r0a // 16, _N // 16, make_body(0, r0a), 0)
    cb.wait()
    lax.fori_loop(r0b // 16, _N // 16, make_body(_BLKE, r0b), 0)

    inv_q = np.float32(1.0 / (1 << _QBITS))

    def red(c, carry):
        chan = c // 10
        chunk = c - chan * 10
        fcnt = zero
        fsum = zero
        for ln in range(16):
            a = acc[pl.ds(ln * _CSTRIDE + chan * _BSTRIDE + chunk * 16, 16)]
            cnt = lax.shift_right_arithmetic(a + (1 << 21), 22)
            rem = a - lax.shift_left(cnt, 22)
            fcnt = fcnt + cnt.astype(jnp.float32)
            fsum = fsum + rem.astype(jnp.float32)
        stage[pl.ds(chan * 512 + chunk * 16, 16)] = fcnt
        stage[pl.ds(chan * 512 + 256 + chunk * 16, 16)] = fsum * inv_q
        return carry

    for k in range(1024 // 16):
        stage[pl.ds(k * 16, 16)] = zero
    lax.fori_loop(0, 20, red, 0)
    pltpu.sync_copy(stage, out_hbm.at[pl.ds(wid * 1024, 1024)])


def _tc_body(p_ref, m_ref, o_ref):
    x = jnp.sum(p_ref[...].reshape(_NW, 1024), axis=0)
    xn = x[0:512].reshape(1, 512)
    xp = x[512:1024].reshape(1, 512)
    ns = jnp.sum(x[0:256])
    ps = jnp.sum(x[512:768])
    xn8 = jnp.broadcast_to(xn, (8, 512))
    t8 = jnp.dot(xn8, m_ref[...], preferred_element_type=jnp.float32,
                 precision=jax.lax.Precision.HIGHEST)
    lu = jnp.sum(t8 * jnp.broadcast_to(xp, (8, 512))) * 0.125
    o_ref[...] = (lu / (ps * ns)).reshape(1, 1)


_finalize = pl.pallas_call(
    _tc_body,
    out_shape=jax.ShapeDtypeStruct((1, 1), jnp.float32),
)


def kernel(similarities_matrix, signs_matrix):
    flip = 1.0 - 2.0 * signs_matrix.astype(jnp.float32)
    v = ((similarities_matrix + 3.0) * flip).reshape(-1)
    partials = _sc_hist(v, _ZEROS)
    out = _finalize(partials, _M_CONST)
    return out.reshape(())
